# baseline (speedup 1.0000x reference)
import jax
import jax.numpy as jnp
from jax.experimental import pallas as pl
from jax.experimental.pallas import tpu as pltpu

NUM_TIMESTEPS = 1000
BETA_START = 0.0001
BETA_END = 0.02

_ROWS = 64


def _body(a_ref, c_ref, x_ref, n_ref, o_ref):
    a = a_ref[...].reshape(_ROWS, 1, 1)
    c = c_ref[...].reshape(_ROWS, 1, 1)
    o_ref[...] = a * x_ref[...] + c * n_ref[...]


def _tables():
    betas = jnp.linspace(BETA_START, BETA_END, NUM_TIMESTEPS, dtype=jnp.float32)
    alphas_cumprod = jnp.cumprod(1.0 - betas, axis=0)
    sac = jnp.sqrt(alphas_cumprod)
    somac = jnp.sqrt(1.0 - alphas_cumprod)
    return sac, somac


def kernel(x_start, t, noise):
    B = x_start.shape[0]
    F = x_start.size // B
    S = F // 128
    x = x_start.reshape(B, S, 128)
    n = noise.reshape(B, S, 128)
    sac, somac = _tables()
    t32 = t.astype(jnp.int32)
    a = jnp.take(sac, t32, axis=0).reshape(B, 1)
    c = jnp.take(somac, t32, axis=0).reshape(B, 1)

    out = pl.pallas_call(
        _body,
        grid=(B // _ROWS,),
        in_specs=[
            pl.BlockSpec((_ROWS, 1), lambda i: (i, 0)),
            pl.BlockSpec((_ROWS, 1), lambda i: (i, 0)),
            pl.BlockSpec((_ROWS, S, 128), lambda i: (i, 0, 0)),
            pl.BlockSpec((_ROWS, S, 128), lambda i: (i, 0, 0)),
        ],
        out_specs=pl.BlockSpec((_ROWS, S, 128), lambda i: (i, 0, 0)),
        out_shape=jax.ShapeDtypeStruct((B, S, 128), jnp.float32),
    )(a, c, x, n)
    return out.reshape(x_start.shape)
